# baseline (speedup 1.0000x reference)
import functools

import jax
import jax.numpy as jnp
from jax import lax
from jax.experimental import pallas as pl
from jax.experimental.pallas import tpu as pltpu
from jax.experimental.pallas import tpu_sc as plsc

N = 524288
V = 100000
F = 16
NC, NS = 2, 16
NW = NC * NS
L = 16

V_PAD = 102400
C1 = 800
ROWS_W = V_PAD // NW
NCH1 = ROWS_W // C1
PTS_W = N // NW
C2 = 512
NCH2 = PTS_W // C2

_mesh = plsc.VectorSubcoreMesh(core_axis_name="c", subcore_axis_name="s")
_params = pltpu.CompilerParams(use_tc_tiling_on_sc=False)


@functools.partial(
    pl.kernel,
    mesh=_mesh,
    compiler_params=_params,
    out_type=jax.ShapeDtypeStruct((V_PAD, F), jnp.float32),
    scratch_types=(
        [pltpu.VMEM((C1,), jnp.int32)] * 6
        + [pltpu.VMEM((C1, F), jnp.float32)] * 6
        + [pltpu.VMEM((C1, F), jnp.float32)] * 2
        + [pltpu.SemaphoreType.DMA] * 6
    ),
)
def _fuse(m0h, m1h, m2h, f0h, f1h, f2h, outh,
          m00, m01, m02, m10, m11, m12,
          r00, r01, r02, r10, r11, r12,
          o0, o1, sm0, sm1, sg0, sg1, so0, so1):
    wid = lax.axis_index("s") * NC + lax.axis_index("c")
    base = wid * ROWS_W
    mh = (m0h, m1h, m2h)
    fh = (f0h, f1h, f2h)
    M = ((m00, m01, m02), (m10, m11, m12))
    R = ((r00, r01, r02), (r10, r11, r12))
    O = (o0, o1)
    SM = (sm0, sm1)
    SG = (sg0, sg1)
    SO = (so0, so1)

    def issue_maps(k, s):
        off = base + k * C1
        for c in range(3):
            pltpu.async_copy(mh[c].at[pl.ds(off, C1)], M[s][c], SM[s])

    def wait_maps(s):
        for c in range(3):
            pltpu.make_async_copy(mh[c].at[pl.ds(0, C1)], M[s][c], SM[s]).wait()

    def issue_gathers(s):
        for c in range(3):
            pltpu.async_copy(fh[c].at[M[s][c]], R[s][c], SG[s])

    def wait_gathers(s):
        for c in range(3):
            pltpu.make_async_copy(outh.at[pl.ds(0, C1)], R[s][c], SG[s]).wait()

    def add(s):
        def add_row(i, c):
            O[s][i, :] = R[s][0][i, :] + R[s][1][i, :] + R[s][2][i, :]
            return c

        lax.fori_loop(0, C1, add_row, 0, unroll=8)

    def issue_out(k, s):
        off = base + k * C1
        pltpu.async_copy(O[s], outh.at[pl.ds(off, C1)], SO[s])

    def wait_out(s):
        pltpu.make_async_copy(outh.at[pl.ds(0, C1)], O[s], SO[s]).wait()

    issue_maps(0, 0)
    wait_maps(0)
    issue_gathers(0)
    issue_maps(1, 1)
    wait_maps(1)
    issue_gathers(1)
    wait_gathers(0)
    add(0)
    issue_out(0, 0)
    issue_maps(2, 0)
    wait_maps(0)
    issue_gathers(0)
    wait_gathers(1)
    add(1)
    issue_out(1, 1)
    issue_maps(3, 1)
    wait_maps(1)
    issue_gathers(1)
    wait_gathers(0)
    wait_out(0)
    add(0)
    issue_out(2, 0)
    wait_gathers(1)
    wait_out(1)
    add(1)
    issue_out(3, 1)
    wait_out(0)
    wait_out(1)


@functools.partial(
    pl.kernel,
    mesh=_mesh,
    compiler_params=_params,
    out_type=jax.ShapeDtypeStruct((N // 8, 8 * F), jnp.float32),
    scratch_types=(
        [pltpu.VMEM((C2,), jnp.int32)] * 6
        + [pltpu.VMEM((C2,), jnp.float32)] * 6
        + [pltpu.VMEM((C2, F), jnp.float32)] * 6
        + [pltpu.VMEM((C2 // 8, 8 * F), jnp.float32)] * 2
        + [pltpu.SemaphoreType.DMA] * 8
    ),
)
def _interp(t0h, t1h, t2h, b0h, b1h, b2h, tabh, outh,
            i00, i01, i02, i10, i11, i12,
            b00, b01, b02, b10, b11, b12,
            r00, r01, r02, r10, r11, r12,
            o0, o1, si0, si1, sb0, sb1, sg0, sg1, so0, so1):
    wid = lax.axis_index("s") * NC + lax.axis_index("c")
    base = wid * PTS_W
    th = (t0h, t1h, t2h)
    bh = (b0h, b1h, b2h)
    I = ((i00, i01, i02), (i10, i11, i12))
    B = ((b00, b01, b02), (b10, b11, b12))
    R = ((r00, r01, r02), (r10, r11, r12))
    O = (o0, o1)
    SI = (si0, si1)
    SB = (sb0, sb1)
    SG = (sg0, sg1)
    SO = (so0, so1)

    def issue_idx(k, s):
        off = base + k * C2
        for c in range(3):
            pltpu.async_copy(th[c].at[pl.ds(off, C2)], I[s][c], SI[s])

    def issue_bary(k, s):
        off = base + k * C2
        for c in range(3):
            pltpu.async_copy(bh[c].at[pl.ds(off, C2)], B[s][c], SB[s])

    def issue_gathers(s):
        for c in range(3):
            pltpu.async_copy(tabh.at[I[s][c]], R[s][c], SG[s])

    def wait_idx(s):
        for c in range(3):
            pltpu.make_async_copy(th[c].at[pl.ds(0, C2)], I[s][c], SI[s]).wait()

    def wait_bary(s):
        for c in range(3):
            pltpu.make_async_copy(bh[c].at[pl.ds(0, C2)], B[s][c], SB[s]).wait()

    def wait_gathers(s):
        for c in range(3):
            pltpu.make_async_copy(tabh.at[pl.ds(0, C2)], R[s][c], SG[s]).wait()

    def wait_out(s):
        pltpu.make_async_copy(outh.at[pl.ds(0, C2 // 8)], O[s], SO[s]).wait()

    def compute(s):
        def group(g, c):
            gbase = g * L
            bv0 = B[s][0][pl.ds(gbase, L)]
            bv1 = B[s][1][pl.ds(gbase, L)]
            bv2 = B[s][2][pl.ds(gbase, L)]
            for p in range(L):
                q = gbase + p
                O[s][2 * g + p // 8, pl.ds((p % 8) * F, F)] = (
                    bv0[p] * R[s][0][q, :] + bv1[p] * R[s][1][q, :]
                    + bv2[p] * R[s][2][q, :])
            return c

        lax.fori_loop(0, C2 // L, group, 0)

    def issue_out(k, s):
        off = base + k * C2
        pltpu.async_copy(O[s], outh.at[pl.ds(off // 8, C2 // 8)], SO[s])

    issue_idx(0, 0)
    issue_bary(0, 0)
    wait_idx(0)
    issue_gathers(0)
    issue_idx(1, 1)
    issue_bary(1, 1)
    wait_idx(1)
    issue_gathers(1)
    wait_gathers(0)
    wait_bary(0)
    compute(0)
    issue_out(0, 0)
    issue_idx(2, 0)
    issue_bary(2, 0)
    wait_idx(0)
    issue_gathers(0)
    wait_gathers(1)
    wait_bary(1)
    compute(1)
    issue_out(1, 1)
    issue_idx(3, 1)
    issue_bary(3, 1)

    def pair(ci, carry):
        k = 2 * ci
        for s in (0, 1):
            kk = k + s
            wait_idx(1 - s)
            issue_gathers(1 - s)
            wait_gathers(s)
            wait_bary(s)
            wait_out(s)
            compute(s)
            issue_out(kk, s)
            issue_idx(kk + 2, s)
            issue_bary(kk + 2, s)
        return carry

    lax.fori_loop(1, NCH2 // 2 - 1, pair, 0)

    wait_idx(1)
    issue_gathers(1)
    wait_gathers(0)
    wait_bary(0)
    wait_out(0)
    compute(0)
    issue_out(NCH2 - 2, 0)
    wait_gathers(1)
    wait_bary(1)
    wait_out(1)
    compute(1)
    issue_out(NCH2 - 1, 1)
    wait_out(0)
    wait_out(1)


def kernel(bary, triangle, feat0, feat1, feat2, map0, map1, map2):
    pad = V_PAD - map0.shape[0]
    zpad = jnp.zeros((pad,), jnp.int32)
    tab = _fuse(jnp.concatenate([map0, zpad]),
                jnp.concatenate([map1, zpad]),
                jnp.concatenate([map2, zpad]),
                feat0, feat1, feat2)
    tri_t = triangle.T
    bary_t = bary.T
    packed = _interp(tri_t[0], tri_t[1], tri_t[2],
                     bary_t[0], bary_t[1], bary_t[2], tab)
    return packed.reshape(N, F)


# speedup vs baseline: 1.0504x; 1.0504x over previous
import functools

import jax
import jax.numpy as jnp
from jax import lax
from jax.experimental import pallas as pl
from jax.experimental.pallas import tpu as pltpu
from jax.experimental.pallas import tpu_sc as plsc

N = 524288
V = 100000
F = 16
NC, NS = 2, 16
NW = NC * NS
L = 16

C1 = 800
ROWS_W = 3200
PTS_W = N // NW
C2 = 512
NCH2 = PTS_W // C2

_mesh = plsc.VectorSubcoreMesh(core_axis_name="c", subcore_axis_name="s")
_params = pltpu.CompilerParams(use_tc_tiling_on_sc=False)


@functools.partial(
    pl.kernel,
    mesh=_mesh,
    compiler_params=_params,
    out_type=jax.ShapeDtypeStruct((V, F), jnp.float32),
    scratch_types=(
        [pltpu.VMEM((C1,), jnp.int32)] * 6
        + [pltpu.VMEM((C1, F), jnp.float32)] * 6
        + [pltpu.VMEM((C1, F), jnp.float32)] * 2
        + [pltpu.SemaphoreType.DMA] * 6
    ),
)
def _fuse(m0h, m1h, m2h, f0h, f1h, f2h, outh,
          m00, m01, m02, m10, m11, m12,
          r00, r01, r02, r10, r11, r12,
          o0, o1, sm0, sm1, sg0, sg1, so0, so1):
    wid = lax.axis_index("s") * NC + lax.axis_index("c")
    base = wid * ROWS_W
    mh = (m0h, m1h, m2h)
    fh = (f0h, f1h, f2h)
    M = ((m00, m01, m02), (m10, m11, m12))
    R = ((r00, r01, r02), (r10, r11, r12))
    O = (o0, o1)
    SM = (sm0, sm1)
    SG = (sg0, sg1)
    SO = (so0, so1)

    def issue_maps(k, s):
        off = base + k * C1
        for c in range(3):
            pltpu.async_copy(mh[c].at[pl.ds(off, C1)], M[s][c], SM[s])

    def wait_maps(s):
        for c in range(3):
            pltpu.make_async_copy(mh[c].at[pl.ds(0, C1)], M[s][c], SM[s]).wait()

    def issue_gathers(s):
        for c in range(3):
            pltpu.async_copy(fh[c].at[M[s][c]], R[s][c], SG[s])

    def wait_gathers(s):
        for c in range(3):
            pltpu.make_async_copy(outh.at[pl.ds(0, C1)], R[s][c], SG[s]).wait()

    def add(s):
        def add_row(i, c):
            O[s][i, :] = R[s][0][i, :] + R[s][1][i, :] + R[s][2][i, :]
            return c

        lax.fori_loop(0, C1, add_row, 0, unroll=8)

    def issue_out(k, s):
        off = base + k * C1
        pltpu.async_copy(O[s], outh.at[pl.ds(off, C1)], SO[s])

    def wait_out(s):
        pltpu.make_async_copy(outh.at[pl.ds(0, C1)], O[s], SO[s]).wait()

    @pl.when(wid < NW - 1)
    def _main():
        issue_maps(0, 0)
        wait_maps(0)
        issue_gathers(0)
        issue_maps(1, 1)
        wait_maps(1)
        issue_gathers(1)
        wait_gathers(0)
        add(0)
        issue_out(0, 0)
        issue_maps(2, 0)
        wait_maps(0)
        issue_gathers(0)
        wait_gathers(1)
        add(1)
        issue_out(1, 1)
        issue_maps(3, 1)
        wait_maps(1)
        issue_gathers(1)
        wait_gathers(0)
        wait_out(0)
        add(0)
        issue_out(2, 0)
        wait_gathers(1)
        wait_out(1)
        add(1)
        issue_out(3, 1)
        wait_out(0)
        wait_out(1)

    @pl.when(wid == NW - 1)
    def _tail():
        issue_maps(0, 0)
        wait_maps(0)
        issue_gathers(0)
        wait_gathers(0)
        add(0)
        issue_out(0, 0)
        wait_out(0)


@functools.partial(
    pl.kernel,
    mesh=_mesh,
    compiler_params=_params,
    out_type=jax.ShapeDtypeStruct((N // 8, 8 * F), jnp.float32),
    scratch_types=(
        [pltpu.VMEM((C2,), jnp.int32)] * 6
        + [pltpu.VMEM((C2,), jnp.float32)] * 6
        + [pltpu.VMEM((C2, F), jnp.float32)] * 6
        + [pltpu.VMEM((C2 // 8, 8 * F), jnp.float32)] * 2
        + [pltpu.SemaphoreType.DMA] * 8
    ),
)
def _interp(t0h, t1h, t2h, b0h, b1h, b2h, tabh, outh,
            i00, i01, i02, i10, i11, i12,
            b00, b01, b02, b10, b11, b12,
            r00, r01, r02, r10, r11, r12,
            o0, o1, si0, si1, sb0, sb1, sg0, sg1, so0, so1):
    wid = lax.axis_index("s") * NC + lax.axis_index("c")
    base = wid * PTS_W
    th = (t0h, t1h, t2h)
    bh = (b0h, b1h, b2h)
    I = ((i00, i01, i02), (i10, i11, i12))
    B = ((b00, b01, b02), (b10, b11, b12))
    R = ((r00, r01, r02), (r10, r11, r12))
    O = (o0, o1)
    SI = (si0, si1)
    SB = (sb0, sb1)
    SG = (sg0, sg1)
    SO = (so0, so1)

    def issue_idx(k, s):
        off = base + k * C2
        for c in range(3):
            pltpu.async_copy(th[c].at[pl.ds(off, C2)], I[s][c], SI[s])

    def issue_bary(k, s):
        off = base + k * C2
        for c in range(3):
            pltpu.async_copy(bh[c].at[pl.ds(off, C2)], B[s][c], SB[s])

    def issue_gathers(s):
        for c in range(3):
            pltpu.async_copy(tabh.at[I[s][c]], R[s][c], SG[s])

    def wait_idx(s):
        for c in range(3):
            pltpu.make_async_copy(th[c].at[pl.ds(0, C2)], I[s][c], SI[s]).wait()

    def wait_bary(s):
        for c in range(3):
            pltpu.make_async_copy(bh[c].at[pl.ds(0, C2)], B[s][c], SB[s]).wait()

    def wait_gathers(s):
        for c in range(3):
            pltpu.make_async_copy(tabh.at[pl.ds(0, C2)], R[s][c], SG[s]).wait()

    def wait_out(s):
        pltpu.make_async_copy(outh.at[pl.ds(0, C2 // 8)], O[s], SO[s]).wait()

    def compute(s):
        def group(g, c):
            gbase = g * L
            bv0 = B[s][0][pl.ds(gbase, L)]
            bv1 = B[s][1][pl.ds(gbase, L)]
            bv2 = B[s][2][pl.ds(gbase, L)]
            for p in range(L):
                q = gbase + p
                O[s][2 * g + p // 8, pl.ds((p % 8) * F, F)] = (
                    bv0[p] * R[s][0][q, :] + bv1[p] * R[s][1][q, :]
                    + bv2[p] * R[s][2][q, :])
            return c

        lax.fori_loop(0, C2 // L, group, 0)

    def issue_out(k, s):
        off = base + k * C2
        pltpu.async_copy(O[s], outh.at[pl.ds(off // 8, C2 // 8)], SO[s])

    issue_idx(0, 0)
    issue_bary(0, 0)
    wait_idx(0)
    issue_gathers(0)
    issue_idx(1, 1)
    issue_bary(1, 1)
    wait_idx(1)
    issue_gathers(1)
    wait_gathers(0)
    wait_bary(0)
    compute(0)
    issue_out(0, 0)
    issue_idx(2, 0)
    issue_bary(2, 0)
    wait_idx(0)
    issue_gathers(0)
    wait_gathers(1)
    wait_bary(1)
    compute(1)
    issue_out(1, 1)
    issue_idx(3, 1)
    issue_bary(3, 1)

    def pair(ci, carry):
        k = 2 * ci
        for s in (0, 1):
            kk = k + s
            wait_idx(1 - s)
            issue_gathers(1 - s)
            wait_gathers(s)
            wait_bary(s)
            wait_out(s)
            compute(s)
            issue_out(kk, s)
            issue_idx(kk + 2, s)
            issue_bary(kk + 2, s)
        return carry

    lax.fori_loop(1, NCH2 // 2 - 1, pair, 0)

    wait_idx(1)
    issue_gathers(1)
    wait_gathers(0)
    wait_bary(0)
    wait_out(0)
    compute(0)
    issue_out(NCH2 - 2, 0)
    wait_gathers(1)
    wait_bary(1)
    wait_out(1)
    compute(1)
    issue_out(NCH2 - 1, 1)
    wait_out(0)
    wait_out(1)


def kernel(bary, triangle, feat0, feat1, feat2, map0, map1, map2):
    tab = _fuse(map0, map1, map2, feat0, feat1, feat2)
    tri_t = triangle.T
    bary_t = bary.T
    packed = _interp(tri_t[0], tri_t[1], tri_t[2],
                     bary_t[0], bary_t[1], bary_t[2], tab)
    return packed.reshape(N, F)


# speedup vs baseline: 1.0527x; 1.0022x over previous
import functools

import jax
import jax.numpy as jnp
from jax import lax
from jax.experimental import pallas as pl
from jax.experimental.pallas import tpu as pltpu
from jax.experimental.pallas import tpu_sc as plsc

N = 524288
V = 100000
F = 16
NC, NS = 2, 16
NW = NC * NS
L = 16

C1 = 800
ROWS_W = 3200
PTS_W = N // NW
C2 = 1024
NCH2 = PTS_W // C2

_mesh = plsc.VectorSubcoreMesh(core_axis_name="c", subcore_axis_name="s")
_params = pltpu.CompilerParams(use_tc_tiling_on_sc=False)


@functools.partial(
    pl.kernel,
    mesh=_mesh,
    compiler_params=_params,
    out_type=jax.ShapeDtypeStruct((V, F), jnp.float32),
    scratch_types=(
        [pltpu.VMEM((C1,), jnp.int32)] * 6
        + [pltpu.VMEM((C1, F), jnp.float32)] * 6
        + [pltpu.VMEM((C1, F), jnp.float32)] * 2
        + [pltpu.SemaphoreType.DMA] * 6
    ),
)
def _fuse(m0h, m1h, m2h, f0h, f1h, f2h, outh,
          m00, m01, m02, m10, m11, m12,
          r00, r01, r02, r10, r11, r12,
          o0, o1, sm0, sm1, sg0, sg1, so0, so1):
    wid = lax.axis_index("s") * NC + lax.axis_index("c")
    base = wid * ROWS_W
    mh = (m0h, m1h, m2h)
    fh = (f0h, f1h, f2h)
    M = ((m00, m01, m02), (m10, m11, m12))
    R = ((r00, r01, r02), (r10, r11, r12))
    O = (o0, o1)
    SM = (sm0, sm1)
    SG = (sg0, sg1)
    SO = (so0, so1)

    def issue_maps(k, s):
        off = base + k * C1
        for c in range(3):
            pltpu.async_copy(mh[c].at[pl.ds(off, C1)], M[s][c], SM[s])

    def wait_maps(s):
        for c in range(3):
            pltpu.make_async_copy(mh[c].at[pl.ds(0, C1)], M[s][c], SM[s]).wait()

    def issue_gathers(s):
        for c in range(3):
            pltpu.async_copy(fh[c].at[M[s][c]], R[s][c], SG[s])

    def wait_gathers(s):
        for c in range(3):
            pltpu.make_async_copy(outh.at[pl.ds(0, C1)], R[s][c], SG[s]).wait()

    def add(s):
        def add_row(i, c):
            O[s][i, :] = R[s][0][i, :] + R[s][1][i, :] + R[s][2][i, :]
            return c

        lax.fori_loop(0, C1, add_row, 0, unroll=8)

    def issue_out(k, s):
        off = base + k * C1
        pltpu.async_copy(O[s], outh.at[pl.ds(off, C1)], SO[s])

    def wait_out(s):
        pltpu.make_async_copy(outh.at[pl.ds(0, C1)], O[s], SO[s]).wait()

    @pl.when(wid < NW - 1)
    def _main():
        issue_maps(0, 0)
        wait_maps(0)
        issue_gathers(0)
        issue_maps(1, 1)
        wait_maps(1)
        issue_gathers(1)
        wait_gathers(0)
        add(0)
        issue_out(0, 0)
        issue_maps(2, 0)
        wait_maps(0)
        issue_gathers(0)
        wait_gathers(1)
        add(1)
        issue_out(1, 1)
        issue_maps(3, 1)
        wait_maps(1)
        issue_gathers(1)
        wait_gathers(0)
        wait_out(0)
        add(0)
        issue_out(2, 0)
        wait_gathers(1)
        wait_out(1)
        add(1)
        issue_out(3, 1)
        wait_out(0)
        wait_out(1)

    @pl.when(wid == NW - 1)
    def _tail():
        issue_maps(0, 0)
        wait_maps(0)
        issue_gathers(0)
        wait_gathers(0)
        add(0)
        issue_out(0, 0)
        wait_out(0)


@functools.partial(
    pl.kernel,
    mesh=_mesh,
    compiler_params=_params,
    out_type=jax.ShapeDtypeStruct((N // 8, 8 * F), jnp.float32),
    scratch_types=(
        [pltpu.VMEM((C2,), jnp.int32)] * 6
        + [pltpu.VMEM((C2,), jnp.float32)] * 6
        + [pltpu.VMEM((C2, F), jnp.float32)] * 6
        + [pltpu.VMEM((C2 // 8, 8 * F), jnp.float32)]
        + [pltpu.SemaphoreType.DMA] * 6
    ),
)
def _interp(t0h, t1h, t2h, b0h, b1h, b2h, tabh, outh,
            i00, i01, i02, i10, i11, i12,
            b00, b01, b02, b10, b11, b12,
            r00, r01, r02, r10, r11, r12,
            o0, si0, si1, sb0, sb1, sg0, sg1):
    wid = lax.axis_index("s") * NC + lax.axis_index("c")
    base = wid * PTS_W
    th = (t0h, t1h, t2h)
    bh = (b0h, b1h, b2h)
    I = ((i00, i01, i02), (i10, i11, i12))
    B = ((b00, b01, b02), (b10, b11, b12))
    R = ((r00, r01, r02), (r10, r11, r12))
    O = (o0, o0)
    SI = (si0, si1)
    SB = (sb0, sb1)
    SG = (sg0, sg1)

    def issue_idx(k, s):
        off = base + k * C2
        for c in range(3):
            pltpu.async_copy(th[c].at[pl.ds(off, C2)], I[s][c], SI[s])

    def issue_bary(k, s):
        off = base + k * C2
        for c in range(3):
            pltpu.async_copy(bh[c].at[pl.ds(off, C2)], B[s][c], SB[s])

    def issue_gathers(s):
        for c in range(3):
            pltpu.async_copy(tabh.at[I[s][c]], R[s][c], SG[s])

    def wait_idx(s):
        for c in range(3):
            pltpu.make_async_copy(th[c].at[pl.ds(0, C2)], I[s][c], SI[s]).wait()

    def wait_bary(s):
        for c in range(3):
            pltpu.make_async_copy(bh[c].at[pl.ds(0, C2)], B[s][c], SB[s]).wait()

    def wait_gathers(s):
        for c in range(3):
            pltpu.make_async_copy(tabh.at[pl.ds(0, C2)], R[s][c], SG[s]).wait()

    def compute(s):
        def group(g, c):
            gbase = g * L
            bv0 = B[s][0][pl.ds(gbase, L)]
            bv1 = B[s][1][pl.ds(gbase, L)]
            bv2 = B[s][2][pl.ds(gbase, L)]
            for p in range(L):
                q = gbase + p
                O[s][2 * g + p // 8, pl.ds((p % 8) * F, F)] = (
                    bv0[p] * R[s][0][q, :] + bv1[p] * R[s][1][q, :]
                    + bv2[p] * R[s][2][q, :])
            return c

        lax.fori_loop(0, C2 // L, group, 0)

    def issue_out(k, s):
        off = base + k * C2
        pltpu.sync_copy(O[s], outh.at[pl.ds(off // 8, C2 // 8)])

    issue_idx(0, 0)
    issue_bary(0, 0)
    wait_idx(0)
    issue_gathers(0)
    issue_idx(1, 1)
    issue_bary(1, 1)
    wait_idx(1)
    issue_gathers(1)
    wait_gathers(0)
    wait_bary(0)
    compute(0)
    issue_out(0, 0)
    issue_idx(2, 0)
    issue_bary(2, 0)
    wait_idx(0)
    issue_gathers(0)
    wait_gathers(1)
    wait_bary(1)
    compute(1)
    issue_out(1, 1)
    issue_idx(3, 1)
    issue_bary(3, 1)

    def pair(ci, carry):
        k = 2 * ci
        for s in (0, 1):
            kk = k + s
            wait_idx(1 - s)
            issue_gathers(1 - s)
            wait_gathers(s)
            wait_bary(s)
            compute(s)
            issue_out(kk, s)
            issue_idx(kk + 2, s)
            issue_bary(kk + 2, s)
        return carry

    lax.fori_loop(1, NCH2 // 2 - 1, pair, 0)

    wait_idx(1)
    issue_gathers(1)
    wait_gathers(0)
    wait_bary(0)
    compute(0)
    issue_out(NCH2 - 2, 0)
    wait_gathers(1)
    wait_bary(1)
    compute(1)
    issue_out(NCH2 - 1, 1)


def kernel(bary, triangle, feat0, feat1, feat2, map0, map1, map2):
    tab = _fuse(map0, map1, map2, feat0, feat1, feat2)
    tri_t = triangle.T
    bary_t = bary.T
    packed = _interp(tri_t[0], tri_t[1], tri_t[2],
                     bary_t[0], bary_t[1], bary_t[2], tab)
    return packed.reshape(N, F)
